# Initial kernel scaffold; baseline (speedup 1.0000x reference)
import jax
import jax.numpy as jnp
from jax.experimental import pallas as pl


def kernel(inputs, edge_index, er_edge_index, conv1_w, conv1_b, conv2_w, conv2_b, bn1_g, bn1_b, bn2_g, bn2_b, Wz, bz, Wr, br, Wh, bh, lin1_w, lin1_b, cls1_w, cls1_b, cls2_w, cls2_b, reg_w, reg_b):
    raise NotImplementedError("write your pallas kernel here")


# speedup vs baseline: 30.7416x; 30.7416x over previous
import functools

import jax
import jax.numpy as jnp
from jax import lax
from jax.experimental import pallas as pl
from jax.experimental.pallas import tpu as pltpu
from jax.experimental.pallas import tpu_sc as plsc

N = 512
EMB = 16
WINDOW = 64
SLIDE = 512


def _build_indicator_mats(row, col):
    E_pad = row.shape[0]
    info = plsc.get_sparse_core_info()
    NC, NS = info.num_cores, info.num_subcores
    rows_per_tile = N // (NC * NS)
    blk = rows_per_tile * N
    mesh = plsc.VectorSubcoreMesh(core_axis_name="c", subcore_axis_name="s")

    @functools.partial(
        pl.kernel,
        mesh=mesh,
        out_type=[
            jax.ShapeDtypeStruct((N * N,), jnp.float32),
            jax.ShapeDtypeStruct((N * N,), jnp.float32),
        ],
        scratch_types=[
            pltpu.VMEM((E_pad,), jnp.int32),
            pltpu.VMEM((E_pad,), jnp.int32),
            pltpu.VMEM((blk,), jnp.float32),
            pltpu.VMEM((blk,), jnp.float32),
        ],
        compiler_params=pltpu.CompilerParams(needs_layout_passes=False),
    )
    def k(row_hbm, col_hbm, ao_hbm, ai_hbm, row_v, col_v, ao_v, ai_v):
        wid = lax.axis_index("s") * NC + lax.axis_index("c")
        lo = wid * rows_per_tile
        hi = lo + rows_per_tile

        zeros16 = jnp.zeros((16,), jnp.float32)

        def init_body(i, carry):
            ao_v[pl.ds(i * 16, 16)] = zeros16
            ai_v[pl.ds(i * 16, 16)] = zeros16
            return carry

        lax.fori_loop(0, blk // 16, init_body, 0)

        pltpu.sync_copy(row_hbm, row_v)
        pltpu.sync_copy(col_hbm, col_v)

        ones16 = jnp.ones((16,), jnp.float32)

        def body(i, carry):
            r = row_v[pl.ds(i * 16, 16)]
            c = col_v[pl.ds(i * 16, 16)]
            mo = (c >= lo) & (c < hi)
            plsc.store_scatter(ao_v, [(c - lo) * N + r], ones16, mask=mo)
            mi = (r >= lo) & (r < hi)
            plsc.store_scatter(ai_v, [(r - lo) * N + c], ones16, mask=mi)
            return carry

        lax.fori_loop(0, E_pad // 16, body, 0)

        pltpu.sync_copy(ao_v, ao_hbm.at[pl.ds(wid * blk, blk)])
        pltpu.sync_copy(ai_v, ai_hbm.at[pl.ds(wid * blk, blk)])

    return k(row, col)


def _recurrence_body(win_ref, ao_ref, ai_ref, w1_ref, b1_ref, g1_ref, be1_ref,
                     w2_ref, b2_ref, g2_ref, be2_ref,
                     wz0_ref, wz1_ref, wz2_ref, bz_ref,
                     wr0_ref, wr1_ref, wr2_ref, br_ref,
                     wh0_ref, wh1_ref, wh2_ref, bh_ref,
                     h_out_ref):
    f32 = jnp.float32
    ao = ao_ref[...]
    ai = ai_ref[...]
    deg_out = jnp.sum(ao, axis=0, keepdims=True)
    deg_in = jnp.sum(ao, axis=1, keepdims=True)
    inv_out = jnp.where(deg_out > 0, 1.0 / deg_out, 0.0)
    inv_in = jnp.where(deg_in > 0, 1.0 / deg_in, 0.0)
    mo = ao * inv_out
    mi = ai * inv_in

    w1 = w1_ref[...]
    w2 = w2_ref[...]

    rows8 = lax.broadcasted_iota(jnp.int32, (8, 16), 0)
    cols16 = lax.broadcasted_iota(jnp.int32, (8, 16), 1)
    s0 = (cols16 == 2 * rows8).astype(f32)
    s1 = (cols16 == 2 * rows8 + 1).astype(f32)

    def encoder(xw):
        ys = [jnp.dot(xw[:, 4 * p:4 * p + 8], w1,
                      preferred_element_type=f32) for p in range(15)]
        y = jnp.concatenate(ys, axis=0) + b1_ref[...]
        y = jnp.maximum(y, 0.0)
        m = jnp.mean(y, axis=0, keepdims=True)
        v = jnp.mean((y - m) * (y - m), axis=0, keepdims=True)
        y = (y - m) * lax.rsqrt(v + 1e-5) * g1_ref[...] + be1_ref[...]
        zs = []
        for q in range(2):
            acc = jnp.zeros((N, 8), f32)
            for kk in range(8):
                blk = y[(4 * q + kk) * N:(4 * q + kk + 1) * N, :]
                acc = acc + jnp.dot(blk, w2[kk], preferred_element_type=f32)
            zs.append(acc)
        z = jnp.concatenate(zs, axis=0) + b2_ref[...]
        z = jnp.maximum(z, 0.0)
        m2 = jnp.mean(z, axis=0, keepdims=True)
        v2 = jnp.mean((z - m2) * (z - m2), axis=0, keepdims=True)
        z = (z - m2) * lax.rsqrt(v2 + 1e-5) * g2_ref[...] + be2_ref[...]
        return (jnp.dot(z[:N], s0, preferred_element_type=f32)
                + jnp.dot(z[N:], s1, preferred_element_type=f32))

    def sigmoid(x):
        return 1.0 / (1.0 + jnp.exp(-x))

    def dconv(xcat, w0, w1g, w2g, b):
        po = jnp.dot(mo, xcat, preferred_element_type=f32)
        pi = jnp.dot(mi, xcat, preferred_element_type=f32)
        return (jnp.dot(xcat, w0, preferred_element_type=f32)
                + jnp.dot(po, w1g, preferred_element_type=f32)
                + jnp.dot(pi, w2g, preferred_element_type=f32) + b)

    h = jnp.zeros((N, EMB), f32)
    for step in range(8):
        xw = win_ref[:, step * WINDOW:(step + 1) * WINDOW]
        x = encoder(xw)
        xcat = jnp.concatenate([x, h], axis=1)
        zg = sigmoid(dconv(xcat, wz0_ref[...], wz1_ref[...], wz2_ref[...],
                           bz_ref[...]))
        rg = sigmoid(dconv(xcat, wr0_ref[...], wr1_ref[...], wr2_ref[...],
                           br_ref[...]))
        xcat2 = jnp.concatenate([x, rg * h], axis=1)
        ht = jnp.tanh(dconv(xcat2, wh0_ref[...], wh1_ref[...], wh2_ref[...],
                            bh_ref[...]))
        h = jnp.maximum(zg * h + (1.0 - zg) * ht, 0.0)
    h_out_ref[...] = h


def _lin1_body(h_ref, w_ref, b_ref, o_ref):
    o_ref[...] = lax.dot_general(
        h_ref[...], w_ref[...], (((1,), (1,)), ((), ())),
        preferred_element_type=jnp.float32) + b_ref[...]


def _heads_body(o_ref, c1w_ref, c1b_ref, c2w_ref, c2b_ref, rw_ref, rb_ref,
                ang_ref, reg_ref):
    out = o_ref[...]
    t1 = lax.dot_general(out, c1w_ref[...], (((1,), (1,)), ((), ())),
                         preferred_element_type=jnp.float32) + c1b_ref[...]
    ang_ref[...] = lax.dot_general(
        t1, c2w_ref[...], (((1,), (1,)), ((), ())),
        preferred_element_type=jnp.float32) + c2b_ref[...]
    reg_ref[...] = lax.dot_general(
        out, rw_ref[...], (((1,), (1,)), ((), ())),
        preferred_element_type=jnp.float32) + rb_ref[...]


def kernel(inputs, edge_index, er_edge_index, conv1_w, conv1_b, conv2_w,
           conv2_b, bn1_g, bn1_b, bn2_g, bn2_b, Wz, bz, Wr, br, Wh, bh,
           lin1_w, lin1_b, cls1_w, cls1_b, cls2_w, cls2_b, reg_w, reg_b):
    f32 = jnp.float32
    E = er_edge_index.shape[1]
    E_pad = ((E + 15) // 16) * 16
    pad = jnp.full((E_pad - E,), N, jnp.int32)
    row = jnp.concatenate([er_edge_index[0].astype(jnp.int32), pad])
    col = jnp.concatenate([er_edge_index[1].astype(jnp.int32), pad])

    ao_flat, ai_flat = _build_indicator_mats(row, col)
    ao = ao_flat.reshape(N, N)
    ai = ai_flat.reshape(N, N)

    win = jnp.concatenate(
        [inputs[:, s:s + WINDOW] for s in range(0, 8 * SLIDE, SLIDE)], axis=1)

    w1 = conv1_w.reshape(32, 8).T
    w2 = jnp.transpose(conv2_w, (2, 1, 0))

    def gates(W):
        return W[0, 0] + W[1, 0], W[0, 1], W[1, 1]

    wz0, wz1, wz2 = gates(Wz)
    wr0, wr1, wr2 = gates(Wr)
    wh0, wh1, wh2 = gates(Wh)

    r2 = lambda a: a.reshape(1, -1)

    h = pl.pallas_call(
        _recurrence_body,
        out_shape=jax.ShapeDtypeStruct((N, EMB), f32),
    )(win, ao, ai, w1, r2(conv1_b), r2(bn1_g), r2(bn1_b),
      w2, r2(conv2_b), r2(bn2_g), r2(bn2_b),
      wz0, wz1, wz2, r2(bz), wr0, wr1, wr2, r2(br),
      wh0, wh1, wh2, r2(bh))

    hid = N * EMB
    hid2 = hid // 2
    h_row = h.reshape(1, hid)
    bj = 512
    out_row = pl.pallas_call(
        _lin1_body,
        grid=(hid2 // bj,),
        in_specs=[
            pl.BlockSpec((1, hid), lambda j: (0, 0)),
            pl.BlockSpec((bj, hid), lambda j: (j, 0)),
            pl.BlockSpec((1, bj), lambda j: (0, j)),
        ],
        out_specs=pl.BlockSpec((1, bj), lambda j: (0, j)),
        out_shape=jax.ShapeDtypeStruct((1, hid2), f32),
    )(h_row, lin1_w, lin1_b.reshape(1, hid2))

    dd, no_ = reg_w.shape[0], reg_w.shape[1]
    rw = reg_w.reshape(dd * no_, hid2)
    ang, regf = pl.pallas_call(
        _heads_body,
        out_shape=[
            jax.ShapeDtypeStruct((1, 8), f32),
            jax.ShapeDtypeStruct((1, dd * no_), f32),
        ],
    )(out_row, cls1_w, r2(cls1_b), cls2_w, r2(cls2_b), rw, r2(reg_b))

    return (er_edge_index, regf.reshape(dd, no_), ang.reshape(8))


# speedup vs baseline: 33.6771x; 1.0955x over previous
import functools

import jax
import jax.numpy as jnp
from jax import lax
from jax.experimental import pallas as pl
from jax.experimental.pallas import tpu as pltpu
from jax.experimental.pallas import tpu_sc as plsc

N = 512
EMB = 16
WINDOW = 64
SLIDE = 512


def _build_indicator_mats(row, col):
    E_pad = row.shape[0]
    iters = E_pad // 16
    info = plsc.get_sparse_core_info()
    n_tiles = info.num_cores * info.num_subcores
    half = n_tiles // 2
    rows_per_tile = N // half
    blk = rows_per_tile * N
    mesh = plsc.VectorSubcoreMesh(core_axis_name="c", subcore_axis_name="s")

    @functools.partial(
        pl.kernel,
        mesh=mesh,
        out_type=[
            jax.ShapeDtypeStruct((N * N,), jnp.float32),
            jax.ShapeDtypeStruct((N * N,), jnp.float32),
        ],
        scratch_types=[
            pltpu.VMEM((E_pad,), jnp.int32),
            pltpu.VMEM((E_pad,), jnp.int32),
            pltpu.VMEM((blk,), jnp.float32),
        ],
        compiler_params=pltpu.CompilerParams(needs_layout_passes=False),
    )
    def k(row_hbm, col_hbm, ao_hbm, ai_hbm, row_v, col_v, a_v):
        wid = lax.axis_index("s") * info.num_cores + lax.axis_index("c")
        on_ao = wid < half
        slot = jnp.where(on_ao, wid, wid - half)
        lo = slot * rows_per_tile
        hi = lo + rows_per_tile

        zeros16 = jnp.zeros((16,), jnp.float32)

        def init_body(i, carry):
            a_v[pl.ds(i * 16, 16)] = zeros16
            return carry

        lax.fori_loop(0, blk // 16, init_body, 0, unroll=4)

        pltpu.sync_copy(row_hbm, row_v)
        pltpu.sync_copy(col_hbm, col_v)

        ones16 = jnp.ones((16,), jnp.float32)

        def body(i, carry):
            r = row_v[pl.ds(i * 16, 16)]
            c = col_v[pl.ds(i * 16, 16)]
            dst = jnp.where(on_ao, c, r)
            src = jnp.where(on_ao, r, c)
            m = (dst >= lo) & (dst < hi)
            plsc.store_scatter(a_v, [(dst - lo) * N + src], ones16, mask=m)
            return carry

        lax.fori_loop(0, iters, body, 0, unroll=4)

        @pl.when(on_ao)
        def _():
            pltpu.sync_copy(a_v, ao_hbm.at[pl.ds(slot * blk, blk)])

        @pl.when(jnp.logical_not(on_ao))
        def _():
            pltpu.sync_copy(a_v, ai_hbm.at[pl.ds(slot * blk, blk)])

    return k(row, col)


def _encoder_body(inp_ref, w1_ref, b1_ref, g1_ref, be1_ref,
                  w2_ref, b2_ref, g2_ref, be2_ref, xs_ref):
    f32 = jnp.float32
    w1 = w1_ref[...]
    w2 = w2_ref[...]

    rows8 = lax.broadcasted_iota(jnp.int32, (8, 16), 0)
    cols16 = lax.broadcasted_iota(jnp.int32, (8, 16), 1)
    s0 = (cols16 == 2 * rows8).astype(f32)
    s1 = (cols16 == 2 * rows8 + 1).astype(f32)

    for step in range(8):
        xw = inp_ref[:, step * SLIDE:step * SLIDE + WINDOW]
        ys = [jnp.dot(xw[:, 4 * p:4 * p + 8], w1,
                      preferred_element_type=f32) for p in range(15)]
        y = jnp.concatenate(ys, axis=0) + b1_ref[...]
        y = jnp.maximum(y, 0.0)
        m = jnp.mean(y, axis=0, keepdims=True)
        v = jnp.mean((y - m) * (y - m), axis=0, keepdims=True)
        y = (y - m) * lax.rsqrt(v + 1e-5) * g1_ref[...] + be1_ref[...]
        zs = []
        for q in range(2):
            acc = jnp.zeros((N, 8), f32)
            for kk in range(8):
                blk = y[(4 * q + kk) * N:(4 * q + kk + 1) * N, :]
                acc = acc + jnp.dot(blk, w2[kk], preferred_element_type=f32)
            zs.append(acc)
        z = jnp.concatenate(zs, axis=0) + b2_ref[...]
        z = jnp.maximum(z, 0.0)
        m2 = jnp.mean(z, axis=0, keepdims=True)
        v2 = jnp.mean((z - m2) * (z - m2), axis=0, keepdims=True)
        z = (z - m2) * lax.rsqrt(v2 + 1e-5) * g2_ref[...] + be2_ref[...]
        xs_ref[:, step * 16:(step + 1) * 16] = (
            jnp.dot(z[:N], s0, preferred_element_type=f32)
            + jnp.dot(z[N:], s1, preferred_element_type=f32))


def _gru_body(xs_ref, ao_ref, ai_ref,
              wz0_ref, wz1_ref, wz2_ref, bz_ref,
              wr0_ref, wr1_ref, wr2_ref, br_ref,
              wh0_ref, wh1_ref, wh2_ref, bh_ref,
              h_out_ref):
    f32 = jnp.float32
    ao = ao_ref[...]
    ai = ai_ref[...]
    deg_out = jnp.sum(ao, axis=0, keepdims=True)
    deg_in = jnp.sum(ao, axis=1, keepdims=True)
    inv_out = jnp.where(deg_out > 0, 1.0 / deg_out, 0.0)
    inv_in = jnp.where(deg_in > 0, 1.0 / deg_in, 0.0)
    mo = ao * inv_out
    mi = ai * inv_in

    def sigmoid(x):
        return 1.0 / (1.0 + jnp.exp(-x))

    def dconv(xcat, w0, w1g, w2g, b):
        po = jnp.dot(mo, xcat, preferred_element_type=f32)
        pi = jnp.dot(mi, xcat, preferred_element_type=f32)
        return (jnp.dot(xcat, w0, preferred_element_type=f32)
                + jnp.dot(po, w1g, preferred_element_type=f32)
                + jnp.dot(pi, w2g, preferred_element_type=f32) + b)

    h = jnp.zeros((N, EMB), f32)
    for step in range(8):
        x = xs_ref[:, step * 16:(step + 1) * 16]
        xcat = jnp.concatenate([x, h], axis=1)
        zg = sigmoid(dconv(xcat, wz0_ref[...], wz1_ref[...], wz2_ref[...],
                           bz_ref[...]))
        rg = sigmoid(dconv(xcat, wr0_ref[...], wr1_ref[...], wr2_ref[...],
                           br_ref[...]))
        xcat2 = jnp.concatenate([x, rg * h], axis=1)
        ht = jnp.tanh(dconv(xcat2, wh0_ref[...], wh1_ref[...], wh2_ref[...],
                            bh_ref[...]))
        h = jnp.maximum(zg * h + (1.0 - zg) * ht, 0.0)
    h_out_ref[...] = h


def _lin1_body(h_ref, w_ref, b_ref, o_ref):
    o_ref[...] = lax.dot_general(
        h_ref[...], w_ref[...], (((1,), (1,)), ((), ())),
        preferred_element_type=jnp.float32) + b_ref[...]


def _heads_body(o_ref, c1w_ref, c1b_ref, c2w_ref, c2b_ref, rw_ref, rb_ref,
                ang_ref, reg_ref):
    out = o_ref[...]
    t1 = lax.dot_general(out, c1w_ref[...], (((1,), (1,)), ((), ())),
                         preferred_element_type=jnp.float32) + c1b_ref[...]
    ang_ref[...] = lax.dot_general(
        t1, c2w_ref[...], (((1,), (1,)), ((), ())),
        preferred_element_type=jnp.float32) + c2b_ref[...]
    reg_ref[...] = lax.dot_general(
        out, rw_ref[...], (((1,), (1,)), ((), ())),
        preferred_element_type=jnp.float32) + rb_ref[...]


def kernel(inputs, edge_index, er_edge_index, conv1_w, conv1_b, conv2_w,
           conv2_b, bn1_g, bn1_b, bn2_g, bn2_b, Wz, bz, Wr, br, Wh, bh,
           lin1_w, lin1_b, cls1_w, cls1_b, cls2_w, cls2_b, reg_w, reg_b):
    f32 = jnp.float32
    E = er_edge_index.shape[1]
    E_pad = ((E + 15) // 16) * 16
    pad = jnp.full((E_pad - E,), N, jnp.int32)
    row = jnp.concatenate([er_edge_index[0].astype(jnp.int32), pad])
    col = jnp.concatenate([er_edge_index[1].astype(jnp.int32), pad])
    ao_flat, ai_flat = _build_indicator_mats(row, col)
    ao = ao_flat.reshape(N, N)
    ai = ai_flat.reshape(N, N)

    w1 = conv1_w.reshape(32, 8).T
    w2 = jnp.transpose(conv2_w, (2, 1, 0))
    r2 = lambda a: a.reshape(1, -1)

    xs = pl.pallas_call(
        _encoder_body,
        out_shape=jax.ShapeDtypeStruct((N, 8 * 16), f32),
    )(inputs, w1, r2(conv1_b), r2(bn1_g), r2(bn1_b),
      w2, r2(conv2_b), r2(bn2_g), r2(bn2_b))

    def gates(W):
        return W[0, 0] + W[1, 0], W[0, 1], W[1, 1]

    wz0, wz1, wz2 = gates(Wz)
    wr0, wr1, wr2 = gates(Wr)
    wh0, wh1, wh2 = gates(Wh)

    h = pl.pallas_call(
        _gru_body,
        out_shape=jax.ShapeDtypeStruct((N, EMB), f32),
    )(xs, ao, ai,
      wz0, wz1, wz2, r2(bz), wr0, wr1, wr2, r2(br),
      wh0, wh1, wh2, r2(bh))

    hid = N * EMB
    hid2 = hid // 2
    h_row = h.reshape(1, hid)
    bj = 512
    out_row = pl.pallas_call(
        _lin1_body,
        grid=(hid2 // bj,),
        in_specs=[
            pl.BlockSpec((1, hid), lambda j: (0, 0)),
            pl.BlockSpec((bj, hid), lambda j: (j, 0)),
            pl.BlockSpec((1, bj), lambda j: (0, j)),
        ],
        out_specs=pl.BlockSpec((1, bj), lambda j: (0, j)),
        out_shape=jax.ShapeDtypeStruct((1, hid2), f32),
    )(h_row, lin1_w, lin1_b.reshape(1, hid2))

    dd, no_ = reg_w.shape[0], reg_w.shape[1]
    rw = reg_w.reshape(dd * no_, hid2)
    ang, regf = pl.pallas_call(
        _heads_body,
        out_shape=[
            jax.ShapeDtypeStruct((1, 8), f32),
            jax.ShapeDtypeStruct((1, dd * no_), f32),
        ],
    )(out_row, cls1_w, r2(cls1_b), cls2_w, r2(cls2_b), rw, r2(reg_b))

    return (er_edge_index, regf.reshape(dd, no_), ang.reshape(8))
